# baseline (speedup 1.0000x reference)
import functools

import jax
import jax.numpy as jnp
import numpy as np
from jax import lax
from jax.experimental import pallas as pl
from jax.experimental.pallas import tpu as pltpu
from jax.experimental.pallas import tpu_sc as plsc

_CUTOFF = 5.0
_NRBF = 16
_WIDTH = _CUTOFF / (_NRBF - 1)
_CENTERS = np.linspace(0.0, _CUTOFF, _NRBF, dtype=np.float32)


_PAIRS_PER_ROW = 64
_BLK_ROWS = 400
_LANES = _PAIRS_PER_ROW * _NRBF


def _rbf_body(d64_ref, d128_ref, e_ref, ct_ref, fij_ref, fcut_ref):
    d = d64_ref[...]
    dh = d.astype(jnp.bfloat16)
    dl = (d - dh.astype(jnp.float32)).astype(jnp.bfloat16)
    dhl = jnp.concatenate([dh, dl], axis=1)
    dexp = jnp.dot(dhl, e_ref[...],
                   preferred_element_type=jnp.float32)
    diff = (dexp - ct_ref[0:1, :]) * np.float32(1.0 / _WIDTH)
    fij_ref[...] = jnp.exp(-0.5 * diff * diff)
    x = d128_ref[...]
    fc = 0.5 * (jnp.cos(x * np.float32(np.pi / _CUTOFF)) + 1.0)
    fcut_ref[...] = jnp.where(x < _CUTOFF, fc, 0.0)


def _rbf_tc(d_flat):
    p = d_flat.shape[0]
    rows64 = p // _PAIRS_PER_ROW
    grid = rows64 // _BLK_ROWS
    blk2 = (_BLK_ROWS * _PAIRS_PER_ROW) // 128

    e_np = np.zeros((64, _LANES), dtype=np.float32)
    e_np[np.arange(_LANES) // _NRBF, np.arange(_LANES)] = 1.0
    e2 = jnp.asarray(np.vstack([e_np, e_np]).astype(jnp.bfloat16))
    ct = jnp.asarray(np.tile(np.tile(_CENTERS, _PAIRS_PER_ROW), (8, 1)))

    d64 = d_flat.reshape(rows64, _PAIRS_PER_ROW)
    d128 = d_flat.reshape(p // 128, 128)

    fij, fcut = pl.pallas_call(
        _rbf_body,
        grid=(grid,),
        in_specs=[
            pl.BlockSpec((_BLK_ROWS, _PAIRS_PER_ROW), lambda i: (i, 0)),
            pl.BlockSpec((blk2, 128), lambda i: (i, 0)),
            pl.BlockSpec((128, _LANES), lambda i: (0, 0)),
            pl.BlockSpec((8, _LANES), lambda i: (0, 0)),
        ],
        out_specs=[
            pl.BlockSpec((_BLK_ROWS, _LANES), lambda i: (i, 0)),
            pl.BlockSpec((blk2, 128), lambda i: (i, 0)),
        ],
        out_shape=[
            jax.ShapeDtypeStruct((rows64, _LANES), jnp.float32),
            jax.ShapeDtypeStruct((p // 128, 128), jnp.float32),
        ],
        compiler_params=pltpu.CompilerParams(
            dimension_semantics=("arbitrary",)),
    )(d64, d128, e2, ct)
    return fij.reshape(p, _NRBF), fcut.reshape(p, 1)


_SC_NW = 32
_SC_BPW = 3200
_SC_GRP = _SC_BPW // 16
_N_ATOMS = 100000
_EMB_D = 32
_MAX_Z = 101
_SC_LAST = _N_ATOMS - (_SC_NW - 1) * _SC_BPW


@functools.cache
def _build_emb_sc():
    @functools.partial(
        pl.kernel,
        out_type=jax.ShapeDtypeStruct((_N_ATOMS * _EMB_D,), jnp.float32),
        mesh=plsc.VectorSubcoreMesh(core_axis_name="c", subcore_axis_name="s"),
        compiler_params=pltpu.CompilerParams(needs_layout_passes=False),
        scratch_types=[
            pltpu.VMEM((_SC_BPW,), jnp.int32),
            pltpu.VMEM((_MAX_Z * _EMB_D,), jnp.float32),
            pltpu.VMEM((_SC_BPW * _EMB_D,), jnp.float32),
        ],
    )
    def _emb_sc(idx_hbm, table_hbm, out_hbm, idx_v, table_v, rows_v):
        wid = lax.axis_index("s") * 2 + lax.axis_index("c")
        base = wid * _SC_BPW
        pltpu.sync_copy(idx_hbm.at[pl.ds(base, _SC_BPW)], idx_v)
        pltpu.sync_copy(table_hbm, table_v)
        iota = lax.iota(jnp.int32, 16)

        def body(g, _):
            z16 = idx_v[pl.ds(g * 16, 16)]
            src = z16 * _EMB_D
            dst = iota * _EMB_D + g * (16 * _EMB_D)
            for j in range(_EMB_D):
                vals = plsc.load_gather(table_v, [src + j])
                plsc.store_scatter(rows_v, [dst + j], vals)
            return 0

        lax.fori_loop(0, _SC_GRP, body, 0)

        @pl.when(wid < _SC_NW - 1)
        def _():
            pltpu.sync_copy(rows_v,
                            out_hbm.at[pl.ds(base * _EMB_D, _SC_BPW * _EMB_D)])

        @pl.when(wid == _SC_NW - 1)
        def _():
            pltpu.sync_copy(rows_v.at[pl.ds(0, _SC_LAST * _EMB_D)],
                            out_hbm.at[pl.ds(base * _EMB_D, _SC_LAST * _EMB_D)])

    return _emb_sc


def _emb_gather(atomic_numbers, embedding_table):
    pad = _SC_NW * _SC_BPW - atomic_numbers.shape[0]
    idx = jnp.concatenate([atomic_numbers, jnp.zeros((pad,), jnp.int32)])
    flat = _build_emb_sc()(idx, embedding_table.reshape(-1))
    return flat.reshape(_N_ATOMS, _EMB_D)


def kernel(d_ij, atomic_numbers, embedding_table):
    f_ij, f_cutoff = _rbf_tc(d_ij.reshape(-1))
    atomic_embedding = _emb_gather(atomic_numbers, embedding_table)
    return (f_ij, f_cutoff, atomic_embedding)


# speedup vs baseline: 8.7786x; 8.7786x over previous
import functools

import jax
import jax.numpy as jnp
import numpy as np
from jax import lax
from jax.experimental import pallas as pl
from jax.experimental.pallas import tpu as pltpu
from jax.experimental.pallas import tpu_sc as plsc

_CUTOFF = 5.0
_NRBF = 16
_INV_W = np.float32(1.0) / np.float32(_CUTOFF / (_NRBF - 1))


_L = 25600


def _rbf_body(d3_ref, d128_ref, fij_ref, fcut_ref):
    d = d3_ref[...].reshape(1, _L)
    db = jnp.broadcast_to(d * _INV_W, (_NRBF, _L))
    k = lax.broadcasted_iota(jnp.int32, (_NRBF, _L), 0).astype(jnp.float32)
    t = db - k
    fij_ref[...] = jnp.exp(-0.5 * t * t)
    x = d128_ref[...]
    fc = 0.5 * (jnp.cos(x * np.float32(np.pi / _CUTOFF)) + 1.0)
    fcut_ref[...] = jnp.where(x < _CUTOFF, fc, 0.0)


def _rbf_tc(d_flat):
    p = d_flat.shape[0]
    grid = p // _L
    r2 = _L // 128

    d3 = d_flat.reshape(grid, 1, _L)
    d128 = d_flat.reshape(p // 128, 128)

    fij_t, fcut = pl.pallas_call(
        _rbf_body,
        grid=(grid,),
        in_specs=[
            pl.BlockSpec((1, 1, _L), lambda i: (i, 0, 0)),
            pl.BlockSpec((r2, 128), lambda i: (i, 0)),
        ],
        out_specs=[
            pl.BlockSpec((_NRBF, _L), lambda i: (0, i)),
            pl.BlockSpec((r2, 128), lambda i: (i, 0)),
        ],
        out_shape=[
            jax.ShapeDtypeStruct((_NRBF, p), jnp.float32),
            jax.ShapeDtypeStruct((p // 128, 128), jnp.float32),
        ],
        compiler_params=pltpu.CompilerParams(
            dimension_semantics=("arbitrary",)),
    )(d3, d128)
    return fij_t.T, fcut.reshape(p, 1)


_SC_NW = 32
_SC_BPW = 3200
_SC_GRP = _SC_BPW // 16
_N_ATOMS = 100000
_EMB_D = 32
_MAX_Z = 101
_SC_NPAD = _SC_NW * _SC_BPW


@functools.cache
def _build_emb_sc():
    @functools.partial(
        pl.kernel,
        out_type=jax.ShapeDtypeStruct((_EMB_D, _SC_NPAD), jnp.float32),
        mesh=plsc.VectorSubcoreMesh(core_axis_name="c", subcore_axis_name="s"),
        compiler_params=pltpu.CompilerParams(needs_layout_passes=False),
        scratch_types=[
            pltpu.VMEM((_SC_BPW,), jnp.int32),
            pltpu.VMEM((_MAX_Z * _EMB_D,), jnp.float32),
            pltpu.VMEM((_EMB_D, _SC_BPW), jnp.float32),
        ],
    )
    def _emb_sc(idx_hbm, table_hbm, out_hbm, idx_v, table_v, rows_v):
        wid = lax.axis_index("s") * 2 + lax.axis_index("c")
        base = wid * _SC_BPW
        pltpu.sync_copy(idx_hbm.at[pl.ds(base, _SC_BPW)], idx_v)
        pltpu.sync_copy(table_hbm, table_v)

        def body(g, _):
            src = idx_v[pl.ds(g * 16, 16)] * _EMB_D
            for j in range(_EMB_D):
                vals = plsc.load_gather(table_v, [src + j])
                rows_v[j, pl.ds(g * 16, 16)] = vals
            return 0

        lax.fori_loop(0, _SC_GRP, body, 0)
        pltpu.sync_copy(rows_v, out_hbm.at[:, pl.ds(base, _SC_BPW)])

    return _emb_sc


def _emb_gather(atomic_numbers, embedding_table):
    pad = _SC_NPAD - atomic_numbers.shape[0]
    idx = jnp.concatenate([atomic_numbers, jnp.zeros((pad,), jnp.int32)])
    emb_t = _build_emb_sc()(idx, embedding_table.reshape(-1))
    return emb_t[:, :_N_ATOMS].T


def kernel(d_ij, atomic_numbers, embedding_table):
    f_ij, f_cutoff = _rbf_tc(d_ij.reshape(-1))
    atomic_embedding = _emb_gather(atomic_numbers, embedding_table)
    return (f_ij, f_cutoff, atomic_embedding)
